# baseline (speedup 1.0000x reference)
import functools
import math

import numpy as np

import jax
import jax.numpy as jnp
from jax import lax
from jax.experimental import pallas as pl
from jax.experimental.pallas import tpu as pltpu
from jax.experimental.pallas import tpu_sc as plsc

N = 10000
E = 320000
F = 128
FP = F // 2

NS = 16
CHUNK = 40
EPT = E // NS
NCHUNK = EPT // CHUNK
K = 5
NBLK = NCHUNK // K
assert K * NBLK == NCHUNK and NBLK % 2 == 0
ROW_STRIDE = 624
ROW_COPY = 640
assert ROW_STRIDE * (NS - 1) + ROW_COPY == N

_BN_SCALE = 1.0 / math.sqrt(1.0 + 1e-3)

_PERM = np.empty(F, np.int32)
for _g in range(4):
    for _j in range(16):
        _PERM[32 * _g + 2 * _j] = 32 * _g + _j
        _PERM[32 * _g + 2 * _j + 1] = 32 * _g + 16 + _j


def _sc_segment_sums(x_packed, idx_all):
    mesh = plsc.VectorSubcoreMesh(core_axis_name="c", subcore_axis_name="s")
    out = jax.ShapeDtypeStruct((N, F), jnp.float32)

    @functools.partial(
        pl.kernel,
        out_type=(out, out),
        mesh=mesh,
        scratch_types=[
            pltpu.VMEM((K, CHUNK), jnp.int32),
            pltpu.VMEM((K, CHUNK), jnp.int32),
            pltpu.VMEM((K, CHUNK), jnp.int32),
            pltpu.VMEM((K, CHUNK), jnp.int32),
            pltpu.VMEM((CHUNK, FP), jnp.int32),
            pltpu.VMEM((CHUNK, FP), jnp.int32),
            pltpu.VMEM((CHUNK, F), jnp.float32),
            pltpu.VMEM((CHUNK, F), jnp.float32),
            pltpu.VMEM_SHARED((N, F), jnp.float32),
            pltpu.SemaphoreType.DMA,
            pltpu.SemaphoreType.DMA,
            pltpu.SemaphoreType.DMA,
            pltpu.SemaphoreType.DMA,
            pltpu.SemaphoreType.DMA,
        ],
        compiler_params=pltpu.CompilerParams(use_tc_tiling_on_sc=False),
    )
    def seg_sum(xp_hbm, idx_hbm, prev_hbm, next_hbm,
                srcb0, srcb1, dstb0, dstb1, pk0, pk1, rows0, rows1, acc,
                semi, semg0, semg1, sems0, sems1):
        cid = lax.axis_index("c")
        sid = lax.axis_index("s")
        row0 = pl.multiple_of(sid * ROW_STRIDE, 8)
        srcb = [srcb0, srcb1]
        dstb = [dstb0, dstb1]
        pk = [pk0, pk1]
        rows = [rows0, rows1]
        semg = [semg0, semg1]
        sems = [sems0, sems1]

        def g_start(mb, k, b):
            pltpu.async_copy(xp_hbm.at[srcb[mb].at[k]], pk[b], semg[b])

        def g_wait(mb, k, b):
            pltpu.make_async_copy(xp_hbm.at[srcb[mb].at[k]], pk[b],
                                  semg[b]).wait()

        def s_start(mb, k, b):
            pltpu.async_copy(rows[b], acc.at[dstb[mb].at[k]], sems[b],
                             add=True)

        def s_wait(mb, k, b):
            pltpu.make_async_copy(rows[b], acc.at[dstb[mb].at[k]],
                                  sems[b]).wait()

        def i_start(mb, m):
            pltpu.async_copy(idx_hbm.at[cid, 1, sid, m], srcb[mb], semi)
            pltpu.async_copy(idx_hbm.at[cid, 0, sid, m], dstb[mb], semi)

        def i_wait(mb, m):
            pltpu.make_async_copy(idx_hbm.at[cid, 1, sid, m], srcb[mb],
                                  semi).wait()
            pltpu.make_async_copy(idx_hbm.at[cid, 0, sid, m], dstb[mb],
                                  semi).wait()

        hi_mask = jnp.full((16,), -65536, jnp.int32)
        sh16 = jnp.full((16,), 16, jnp.int32)

        def convert(b):
            src = pk[b]
            dst = rows[b]

            def row_body(r, carry):
                for g in range(4):
                    v = src[r, pl.ds(16 * g, 16)]
                    lo = lax.bitcast_convert_type(v << sh16, jnp.float32)
                    hi = lax.bitcast_convert_type(v & hi_mask, jnp.float32)
                    dst[r, pl.ds(32 * g, 16)] = lo
                    dst[r, pl.ds(32 * g + 16, 16)] = hi
                return carry

            lax.fori_loop(0, CHUNK, row_body, 0)

        pltpu.sync_copy(idx_hbm.at[cid, 1, sid, 0], srcb[0])
        pltpu.sync_copy(idx_hbm.at[cid, 0, sid, 0], dstb[0])
        i_start(1, 1)
        g_start(0, 0, 0)

        zv = jnp.zeros((16,), jnp.float32)

        def zero_row(r, carry):
            for c in range(F // 16):
                rows1[r, pl.ds(c * 16, 16)] = zv
            return carry

        lax.fori_loop(0, CHUNK, zero_row, 0)
        for ii in range(ROW_COPY // CHUNK):
            pltpu.async_copy(rows1, acc.at[pl.ds(row0 + ii * CHUNK, CHUNK)],
                             sems0)
        for ii in range(ROW_COPY // CHUNK):
            pltpu.make_async_copy(rows1,
                                  acc.at[pl.ds(row0 + ii * CHUNK, CHUNK)],
                                  sems0).wait()
        plsc.subcore_barrier()

        def block_pair(i, carry):
            for half in (0, 1):
                m = 2 * i + half
                for k in range(K):
                    b = (half + k) % 2
                    nb = 1 - b
                    g_wait(half, k, b)
                    if k < K - 1:
                        g_start(half, k + 1, nb)
                    else:
                        if half == 0:
                            i_wait(1, m + 1)
                            g_start(1, 0, nb)
                        else:
                            @pl.when(i < NBLK // 2 - 1)
                            def _():
                                i_wait(0, m + 1)
                                g_start(0, 0, nb)
                    if k >= 2:
                        s_wait(half, k - 2, b)
                    elif k == 1:
                        if half == 0:
                            @pl.when(i >= 1)
                            def _():
                                s_wait(1, K - 1, b)
                                i_start(1, m + 1)
                        else:
                            s_wait(0, K - 1, b)

                            @pl.when(i < NBLK // 2 - 1)
                            def _():
                                i_start(0, m + 1)
                    else:
                        if half == 0:
                            @pl.when(i >= 1)
                            def _():
                                s_wait(1, K - 2, b)
                        else:
                            s_wait(0, K - 2, b)
                    convert(b)
                    s_start(half, k, b)
            return carry

        lax.fori_loop(0, NBLK // 2, block_pair, 0)
        s_wait(1, K - 2, 0)
        s_wait(1, K - 1, 1)
        plsc.subcore_barrier()

        @pl.when(cid == 0)
        def _():
            pltpu.sync_copy(acc.at[pl.ds(row0, ROW_COPY)],
                            prev_hbm.at[pl.ds(row0, ROW_COPY)])

        @pl.when(cid == 1)
        def _():
            pltpu.sync_copy(acc.at[pl.ds(row0, ROW_COPY)],
                            next_hbm.at[pl.ds(row0, ROW_COPY)])

    return seg_sum(x_packed, idx_all)


def _tc_recurrent(x, grk, gb1):
    R = 2000

    def body(x_ref, grk_ref, gb1_ref, o_ref):
        o_ref[...] = jnp.dot(x_ref[...], grk_ref[...],
                             preferred_element_type=jnp.float32) + gb1_ref[...]

    return pl.pallas_call(
        body,
        grid=(N // R,),
        in_specs=[
            pl.BlockSpec((R, F), lambda i: (i, 0)),
            pl.BlockSpec((F, 3 * F), lambda i: (0, 0)),
            pl.BlockSpec((3 * F,), lambda i: (0,)),
        ],
        out_specs=pl.BlockSpec((R, 3 * F), lambda i: (i, 0)),
        out_shape=jax.ShapeDtypeStruct((N, 3 * F), jnp.float32),
    )(x, grk, gb1)


def _tc_dense(x, psum, nsum, mi, wNext, wPrev, bvec, gamma, beta, gk, gb0):
    R = 1000

    def body(x_ref, p_ref, n_ref, mi_ref, wn_ref, wp_ref, b_ref, g_ref,
             be_ref, gk_ref, gb0_ref, o_ref):
        xb = x_ref[...]
        aggre = jnp.dot(n_ref[...], wn_ref[...],
                        preferred_element_type=jnp.float32)
        aggre = aggre + jnp.dot(p_ref[...], wp_ref[...],
                                preferred_element_type=jnp.float32)
        aggre = aggre + b_ref[...] + xb
        a = jnp.maximum(aggre, 0.0)
        a = a * (g_ref[...] * _BN_SCALE) + be_ref[...]
        mx = jnp.dot(a, gk_ref[...], preferred_element_type=jnp.float32)
        mx = mx + gb0_ref[...]
        mi = mi_ref[...]
        z = jax.nn.sigmoid(mx[:, 0:F] + mi[:, 0:F])
        r = jax.nn.sigmoid(mx[:, F:2 * F] + mi[:, F:2 * F])
        hh = jnp.tanh(mx[:, 2 * F:] + r * mi[:, 2 * F:])
        o_ref[...] = z * xb + (1.0 - z) * hh

    def full(shape):
        return pl.BlockSpec(shape, lambda i: (0,) * len(shape))

    return pl.pallas_call(
        body,
        grid=(N // R,),
        in_specs=[
            pl.BlockSpec((R, F), lambda i: (i, 0)),
            pl.BlockSpec((R, F), lambda i: (i, 0)),
            pl.BlockSpec((R, F), lambda i: (i, 0)),
            pl.BlockSpec((R, 3 * F), lambda i: (i, 0)),
            full((F, F)),
            full((F, F)),
            full((F,)),
            full((F,)),
            full((F,)),
            full((F, 3 * F)),
            full((3 * F,)),
        ],
        out_specs=pl.BlockSpec((R, F), lambda i: (i, 0)),
        out_shape=jax.ShapeDtypeStruct((N, F), jnp.float32),
    )(x, psum, nsum, mi, wNext, wPrev, bvec, gamma, beta, gk, gb0)


def kernel(x, pairsPrev, pairsNext, kmers, wNext, wPrev, b, gamma, beta,
           gru_kernel, gru_rkernel, gru_bias):
    idx_all = jnp.stack([pairsPrev, pairsNext]).transpose(0, 2, 1)
    idx_all = idx_all.reshape(2, 2, NS, NBLK, K, CHUNK)
    x_packed = jax.lax.bitcast_convert_type(
        x[:, _PERM].astype(jnp.bfloat16).reshape(N, FP, 2), jnp.int32)
    mi = _tc_recurrent(x, gru_rkernel, gru_bias[1])
    psum, nsum = _sc_segment_sums(x_packed, idx_all)
    return _tc_dense(x, psum, nsum, mi, wNext, wPrev, b.reshape(F), gamma,
                     beta, gru_kernel, gru_bias[0])


# speedup vs baseline: 1.3572x; 1.3572x over previous
import functools
import math

import jax
import jax.numpy as jnp
from jax import lax
from jax.experimental import pallas as pl
from jax.experimental.pallas import tpu as pltpu
from jax.experimental.pallas import tpu_sc as plsc

N = 10000
E = 320000
F = 128

NS = 16
CHUNK = 80
EPT = E // NS
NCHUNK = EPT // CHUNK
K = 5
NBLK = NCHUNK // K
assert K * NBLK == NCHUNK and NBLK % 2 == 0
ROW_STRIDE = 624
ROW_COPY = 640
assert ROW_STRIDE * (NS - 1) + ROW_COPY == N

_BN_SCALE = 1.0 / math.sqrt(1.0 + 1e-3)


def _sc_segment_sums(x, idx_all):
    mesh = plsc.VectorSubcoreMesh(core_axis_name="c", subcore_axis_name="s")
    out = jax.ShapeDtypeStruct((N, F), jnp.float32)

    @functools.partial(
        pl.kernel,
        out_type=(out, out),
        mesh=mesh,
        scratch_types=[
            pltpu.VMEM((K, CHUNK), jnp.int32),
            pltpu.VMEM((K, CHUNK), jnp.int32),
            pltpu.VMEM((K, CHUNK), jnp.int32),
            pltpu.VMEM((K, CHUNK), jnp.int32),
            pltpu.VMEM((CHUNK, F), jnp.float32),
            pltpu.VMEM((CHUNK, F), jnp.float32),
            pltpu.VMEM_SHARED((N, F), jnp.float32),
            pltpu.SemaphoreType.DMA,
            pltpu.SemaphoreType.DMA,
            pltpu.SemaphoreType.DMA,
            pltpu.SemaphoreType.DMA,
            pltpu.SemaphoreType.DMA,
        ],
    )
    def seg_sum(x_hbm, idx_hbm, prev_hbm, next_hbm,
                srcb0, srcb1, dstb0, dstb1, rows0, rows1, acc,
                semi, semg0, semg1, sems0, sems1):
        cid = lax.axis_index("c")
        sid = lax.axis_index("s")
        row0 = pl.multiple_of(sid * ROW_STRIDE, 8)
        srcb = [srcb0, srcb1]
        dstb = [dstb0, dstb1]
        rows = [rows0, rows1]
        semg = [semg0, semg1]
        sems = [sems0, sems1]

        def g_start(mb, k, b):
            pltpu.async_copy(x_hbm.at[srcb[mb].at[k]], rows[b], semg[b])

        def g_wait(mb, k, b):
            pltpu.make_async_copy(x_hbm.at[srcb[mb].at[k]], rows[b],
                                  semg[b]).wait()

        def s_start(mb, k, b):
            pltpu.async_copy(rows[b], acc.at[dstb[mb].at[k]], sems[b],
                             add=True)

        def s_wait(mb, k, b):
            pltpu.make_async_copy(rows[b], acc.at[dstb[mb].at[k]],
                                  sems[b]).wait()

        def i_start(mb, m):
            pltpu.async_copy(idx_hbm.at[cid, 1, sid, m], srcb[mb], semi)
            pltpu.async_copy(idx_hbm.at[cid, 0, sid, m], dstb[mb], semi)

        def i_wait(mb, m):
            pltpu.make_async_copy(idx_hbm.at[cid, 1, sid, m], srcb[mb],
                                  semi).wait()
            pltpu.make_async_copy(idx_hbm.at[cid, 0, sid, m], dstb[mb],
                                  semi).wait()

        pltpu.sync_copy(idx_hbm.at[cid, 1, sid, 0], srcb[0])
        pltpu.sync_copy(idx_hbm.at[cid, 0, sid, 0], dstb[0])
        i_start(1, 1)
        g_start(0, 0, 0)

        zv = jnp.zeros((16,), jnp.float32)

        def zero_row(r, carry):
            for c in range(F // 16):
                rows1[r, pl.ds(c * 16, 16)] = zv
            return carry

        lax.fori_loop(0, CHUNK, zero_row, 0)
        for ii in range(ROW_COPY // CHUNK):
            pltpu.async_copy(rows1, acc.at[pl.ds(row0 + ii * CHUNK, CHUNK)],
                             sems0)
        for ii in range(ROW_COPY // CHUNK):
            pltpu.make_async_copy(rows1,
                                  acc.at[pl.ds(row0 + ii * CHUNK, CHUNK)],
                                  sems0).wait()
        plsc.subcore_barrier()

        def block_pair(i, carry):
            for half in (0, 1):
                m = 2 * i + half
                for k in range(K):
                    b = (half + k) % 2
                    nb = 1 - b
                    g_wait(half, k, b)
                    s_start(half, k, b)
                    if k == 0:
                        if half == 0:
                            @pl.when(i >= 1)
                            def _():
                                s_wait(1, K - 1, nb)
                                i_start(1, m + 1)
                        else:
                            s_wait(0, K - 1, nb)

                            @pl.when(i < NBLK // 2 - 1)
                            def _():
                                i_start(0, m + 1)
                    else:
                        s_wait(half, k - 1, nb)
                    if k < K - 1:
                        g_start(half, k + 1, nb)
                    else:
                        if half == 0:
                            i_wait(1, m + 1)
                            g_start(1, 0, nb)
                        else:
                            @pl.when(i < NBLK // 2 - 1)
                            def _():
                                i_wait(0, m + 1)
                                g_start(0, 0, nb)
            return carry

        lax.fori_loop(0, NBLK // 2, block_pair, 0)
        s_wait(1, K - 1, 1)
        plsc.subcore_barrier()

        @pl.when(cid == 0)
        def _():
            pltpu.sync_copy(acc.at[pl.ds(row0, ROW_COPY)],
                            prev_hbm.at[pl.ds(row0, ROW_COPY)])

        @pl.when(cid == 1)
        def _():
            pltpu.sync_copy(acc.at[pl.ds(row0, ROW_COPY)],
                            next_hbm.at[pl.ds(row0, ROW_COPY)])

    return seg_sum(x, idx_all)


def _tc_recurrent(x, grk, gb1):
    R = 2000

    def body(x_ref, grk_ref, gb1_ref, o_ref):
        o_ref[...] = jnp.dot(x_ref[...], grk_ref[...],
                             preferred_element_type=jnp.float32) + gb1_ref[...]

    return pl.pallas_call(
        body,
        grid=(N // R,),
        in_specs=[
            pl.BlockSpec((R, F), lambda i: (i, 0)),
            pl.BlockSpec((F, 3 * F), lambda i: (0, 0)),
            pl.BlockSpec((3 * F,), lambda i: (0,)),
        ],
        out_specs=pl.BlockSpec((R, 3 * F), lambda i: (i, 0)),
        out_shape=jax.ShapeDtypeStruct((N, 3 * F), jnp.float32),
    )(x, grk, gb1)


def _tc_dense(x, psum, nsum, wNext, wPrev, bvec, gamma, beta, gk, gb0,
              grk, gb1):
    R = 1000

    def body(x_ref, p_ref, n_ref, wn_ref, wp_ref, b_ref, g_ref,
             be_ref, gk_ref, gb0_ref, grk_ref, gb1_ref, o_ref):
        xb = x_ref[...]
        aggre = jnp.dot(n_ref[...], wn_ref[...],
                        preferred_element_type=jnp.float32)
        aggre = aggre + jnp.dot(p_ref[...], wp_ref[...],
                                preferred_element_type=jnp.float32)
        aggre = aggre + b_ref[...] + xb
        a = jnp.maximum(aggre, 0.0)
        a = a * (g_ref[...] * _BN_SCALE) + be_ref[...]
        mx = jnp.dot(a, gk_ref[...], preferred_element_type=jnp.float32)
        mx = mx + gb0_ref[...]
        mi = jnp.dot(xb, grk_ref[...], preferred_element_type=jnp.float32)
        mi = mi + gb1_ref[...]
        z = jax.nn.sigmoid(mx[:, 0:F] + mi[:, 0:F])
        r = jax.nn.sigmoid(mx[:, F:2 * F] + mi[:, F:2 * F])
        hh = jnp.tanh(mx[:, 2 * F:] + r * mi[:, 2 * F:])
        o_ref[...] = z * xb + (1.0 - z) * hh

    def full(shape):
        return pl.BlockSpec(shape, lambda i: (0,) * len(shape))

    return pl.pallas_call(
        body,
        grid=(N // R,),
        in_specs=[
            pl.BlockSpec((R, F), lambda i: (i, 0)),
            pl.BlockSpec((R, F), lambda i: (i, 0)),
            pl.BlockSpec((R, F), lambda i: (i, 0)),
            full((F, F)),
            full((F, F)),
            full((F,)),
            full((F,)),
            full((F,)),
            full((F, 3 * F)),
            full((3 * F,)),
            full((F, 3 * F)),
            full((3 * F,)),
        ],
        out_specs=pl.BlockSpec((R, F), lambda i: (i, 0)),
        out_shape=jax.ShapeDtypeStruct((N, F), jnp.float32),
    )(x, psum, nsum, wNext, wPrev, bvec, gamma, beta, gk, gb0, grk, gb1)


def kernel(x, pairsPrev, pairsNext, kmers, wNext, wPrev, b, gamma, beta,
           gru_kernel, gru_rkernel, gru_bias):
    idx_all = jnp.stack([pairsPrev, pairsNext]).transpose(0, 2, 1)
    idx_all = idx_all.reshape(2, 2, NS, NBLK, K, CHUNK)
    psum, nsum = _sc_segment_sums(x, idx_all)
    return _tc_dense(x, psum, nsum, wNext, wPrev, b.reshape(F), gamma,
                     beta, gru_kernel, gru_bias[0], gru_rkernel, gru_bias[1])


# speedup vs baseline: 1.3639x; 1.0049x over previous
import functools
import math

import jax
import jax.numpy as jnp
from jax import lax
from jax.experimental import pallas as pl
from jax.experimental.pallas import tpu as pltpu
from jax.experimental.pallas import tpu_sc as plsc

N = 10000
E = 320000
F = 128

NS = 16
CHUNK = 80
EPT = E // NS
NCHUNK = EPT // CHUNK
K = 5
NBLK = NCHUNK // K
assert K * NBLK == NCHUNK and NBLK % 2 == 0
ROW_STRIDE = 624
ROW_COPY = 640
assert ROW_STRIDE * (NS - 1) + ROW_COPY == N

_BN_SCALE = 1.0 / math.sqrt(1.0 + 1e-3)


def _sc_segment_sums(x, idx_all):
    mesh = plsc.VectorSubcoreMesh(core_axis_name="c", subcore_axis_name="s")
    out = jax.ShapeDtypeStruct((N, F), jnp.float32)

    @functools.partial(
        pl.kernel,
        out_type=(out, out),
        mesh=mesh,
        scratch_types=[
            pltpu.VMEM((K, CHUNK), jnp.int32),
            pltpu.VMEM((K, CHUNK), jnp.int32),
            pltpu.VMEM((K, CHUNK), jnp.int32),
            pltpu.VMEM((K, CHUNK), jnp.int32),
            pltpu.VMEM((CHUNK, F), jnp.float32),
            pltpu.VMEM((CHUNK, F), jnp.float32),
            pltpu.VMEM_SHARED((N, F), jnp.float32),
            pltpu.SemaphoreType.DMA,
            pltpu.SemaphoreType.DMA,
            pltpu.SemaphoreType.DMA,
            pltpu.SemaphoreType.DMA,
            pltpu.SemaphoreType.DMA,
        ],
    )
    def seg_sum(x_hbm, idx_hbm, prev_hbm, next_hbm,
                srcb0, srcb1, dstb0, dstb1, rows0, rows1, acc,
                semi, semg0, semg1, sems0, sems1):
        cid = lax.axis_index("c")
        sid = lax.axis_index("s")
        row0 = pl.multiple_of(sid * ROW_STRIDE, 8)
        srcb = [srcb0, srcb1]
        dstb = [dstb0, dstb1]
        rows = [rows0, rows1]
        semg = [semg0, semg1]
        sems = [sems0, sems1]

        def g_start(mb, k, b):
            pltpu.async_copy(x_hbm.at[srcb[mb].at[k]], rows[b], semg[b])

        def g_wait(mb, k, b):
            pltpu.make_async_copy(x_hbm.at[srcb[mb].at[k]], rows[b],
                                  semg[b]).wait()

        def s_start(mb, k, b):
            pltpu.async_copy(rows[b], acc.at[dstb[mb].at[k]], sems[b],
                             add=True)

        def s_wait(mb, k, b):
            pltpu.make_async_copy(rows[b], acc.at[dstb[mb].at[k]],
                                  sems[b]).wait()

        def i_start(mb, m):
            pltpu.async_copy(idx_hbm.at[cid, 1, sid, m], srcb[mb], semi)
            pltpu.async_copy(idx_hbm.at[cid, 0, sid, m], dstb[mb], semi)

        def i_wait(mb, m):
            pltpu.make_async_copy(idx_hbm.at[cid, 1, sid, m], srcb[mb],
                                  semi).wait()
            pltpu.make_async_copy(idx_hbm.at[cid, 0, sid, m], dstb[mb],
                                  semi).wait()

        pltpu.sync_copy(idx_hbm.at[cid, 1, sid, 0], srcb[0])
        pltpu.sync_copy(idx_hbm.at[cid, 0, sid, 0], dstb[0])
        i_start(1, 1)
        g_start(0, 0, 0)

        zv = jnp.zeros((16,), jnp.float32)

        def zero_row(r, carry):
            for c in range(F // 16):
                rows1[r, pl.ds(c * 16, 16)] = zv
            return carry

        lax.fori_loop(0, CHUNK, zero_row, 0)
        for ii in range(ROW_COPY // CHUNK):
            pltpu.async_copy(rows1, acc.at[pl.ds(row0 + ii * CHUNK, CHUNK)],
                             sems0)
        for ii in range(ROW_COPY // CHUNK):
            pltpu.make_async_copy(rows1,
                                  acc.at[pl.ds(row0 + ii * CHUNK, CHUNK)],
                                  sems0).wait()
        plsc.subcore_barrier()

        def block_pair(i, carry):
            for half in (0, 1):
                m = 2 * i + half
                for k in range(K):
                    b = (half + k) % 2
                    nb = 1 - b
                    g_wait(half, k, b)
                    s_start(half, k, b)
                    if k == 0:
                        if half == 0:
                            @pl.when(i >= 1)
                            def _():
                                s_wait(1, K - 1, nb)
                                i_start(1, m + 1)
                        else:
                            s_wait(0, K - 1, nb)

                            @pl.when(i < NBLK // 2 - 1)
                            def _():
                                i_start(0, m + 1)
                    else:
                        s_wait(half, k - 1, nb)
                    if k < K - 1:
                        g_start(half, k + 1, nb)
                    else:
                        if half == 0:
                            i_wait(1, m + 1)
                            g_start(1, 0, nb)
                        else:
                            @pl.when(i < NBLK // 2 - 1)
                            def _():
                                i_wait(0, m + 1)
                                g_start(0, 0, nb)
            return carry

        lax.fori_loop(0, NBLK // 2, block_pair, 0)
        s_wait(1, K - 1, 1)
        plsc.subcore_barrier()

        @pl.when(cid == 0)
        def _():
            pltpu.sync_copy(acc.at[pl.ds(row0, ROW_COPY)],
                            prev_hbm.at[pl.ds(row0, ROW_COPY)])

        @pl.when(cid == 1)
        def _():
            pltpu.sync_copy(acc.at[pl.ds(row0, ROW_COPY)],
                            next_hbm.at[pl.ds(row0, ROW_COPY)])

    return seg_sum(x, idx_all)


def _tc_recurrent(x, grk, gb1):
    R = 2000

    def body(x_ref, grk_ref, gb1_ref, o_ref):
        o_ref[...] = jnp.dot(x_ref[...], grk_ref[...],
                             preferred_element_type=jnp.float32) + gb1_ref[...]

    return pl.pallas_call(
        body,
        grid=(N // R,),
        in_specs=[
            pl.BlockSpec((R, F), lambda i: (i, 0)),
            pl.BlockSpec((F, 3 * F), lambda i: (0, 0)),
            pl.BlockSpec((3 * F,), lambda i: (0,)),
        ],
        out_specs=pl.BlockSpec((R, 3 * F), lambda i: (i, 0)),
        out_shape=jax.ShapeDtypeStruct((N, 3 * F), jnp.float32),
    )(x, grk, gb1)


def _tc_dense(x, psum, nsum, wNext, wPrev, bvec, gamma, beta, gk, gb0,
              grk, gb1):
    R = 2000

    def body(x_ref, p_ref, n_ref, wn_ref, wp_ref, b_ref, g_ref,
             be_ref, gk_ref, gb0_ref, grk_ref, gb1_ref, o_ref):
        xb = x_ref[...]
        aggre = jnp.dot(n_ref[...], wn_ref[...],
                        preferred_element_type=jnp.float32)
        aggre = aggre + jnp.dot(p_ref[...], wp_ref[...],
                                preferred_element_type=jnp.float32)
        aggre = aggre + b_ref[...] + xb
        a = jnp.maximum(aggre, 0.0)
        a = a * (g_ref[...] * _BN_SCALE) + be_ref[...]
        mx = jnp.dot(a, gk_ref[...], preferred_element_type=jnp.float32)
        mx = mx + gb0_ref[...]
        mi = jnp.dot(xb, grk_ref[...], preferred_element_type=jnp.float32)
        mi = mi + gb1_ref[...]
        z = jax.nn.sigmoid(mx[:, 0:F] + mi[:, 0:F])
        r = jax.nn.sigmoid(mx[:, F:2 * F] + mi[:, F:2 * F])
        hh = jnp.tanh(mx[:, 2 * F:] + r * mi[:, 2 * F:])
        o_ref[...] = z * xb + (1.0 - z) * hh

    def full(shape):
        return pl.BlockSpec(shape, lambda i: (0,) * len(shape))

    return pl.pallas_call(
        body,
        grid=(N // R,),
        in_specs=[
            pl.BlockSpec((R, F), lambda i: (i, 0)),
            pl.BlockSpec((R, F), lambda i: (i, 0)),
            pl.BlockSpec((R, F), lambda i: (i, 0)),
            full((F, F)),
            full((F, F)),
            full((F,)),
            full((F,)),
            full((F,)),
            full((F, 3 * F)),
            full((3 * F,)),
            full((F, 3 * F)),
            full((3 * F,)),
        ],
        out_specs=pl.BlockSpec((R, F), lambda i: (i, 0)),
        out_shape=jax.ShapeDtypeStruct((N, F), jnp.float32),
    )(x, psum, nsum, wNext, wPrev, bvec, gamma, beta, gk, gb0, grk, gb1)


def kernel(x, pairsPrev, pairsNext, kmers, wNext, wPrev, b, gamma, beta,
           gru_kernel, gru_rkernel, gru_bias):
    idx_all = jnp.stack([pairsPrev, pairsNext]).transpose(0, 2, 1)
    idx_all = idx_all.reshape(2, 2, NS, NBLK, K, CHUNK)
    psum, nsum = _sc_segment_sums(x, idx_all)
    return _tc_dense(x, psum, nsum, wNext, wPrev, b.reshape(F), gamma,
                     beta, gru_kernel, gru_bias[0], gru_rkernel, gru_bias[1])
